# Initial kernel scaffold; baseline (speedup 1.0000x reference)
import jax
import jax.numpy as jnp
from jax.experimental import pallas as pl


def kernel(node_feats, edge_feats, params, edge_index):
    raise NotImplementedError("write your pallas kernel here")


# speedup vs baseline: 1.1195x; 1.1195x over previous
import functools

import jax
import jax.numpy as jnp
from jax.experimental import pallas as pl

EPS = 1e-5


def _fold(p):
    s1 = p['g1'] / jnp.sqrt(p['v1'] + EPS)
    t1 = p['be1'] - p['m1'] * s1
    W1 = p['W1'] * s1[None, :]
    c1 = p['b1'] * s1 + t1
    s2 = p['g2'] / jnp.sqrt(p['v2'] + EPS)
    t2 = p['be2'] - p['m2'] * s2
    W2 = p['W2'] * s2[None, :]
    c2 = p['b2'] * s2 + t2
    return W1, c1, W2, c2


def _silu(x):
    return x * jax.nn.sigmoid(x)


def _tables_body(nf_ref, w1a_ref, w1b_ref, c1_ref, psrc_ref, pdst_ref):
    nf = nf_ref[...]
    psrc_ref[...] = jnp.dot(nf, w1a_ref[...], preferred_element_type=jnp.float32)
    pdst_ref[...] = jnp.dot(nf, w1b_ref[...], preferred_element_type=jnp.float32) + c1_ref[...]


def _node_tables(nf, w1a, w1b, c1, blk):
    n, d = nf.shape
    grid = n // blk
    return pl.pallas_call(
        _tables_body,
        grid=(grid,),
        in_specs=[
            pl.BlockSpec((blk, d), lambda i: (i, 0)),
            pl.BlockSpec((d, d), lambda i: (0, 0)),
            pl.BlockSpec((d, d), lambda i: (0, 0)),
            pl.BlockSpec((1, d), lambda i: (0, 0)),
        ],
        out_specs=[
            pl.BlockSpec((blk, d), lambda i: (i, 0)),
            pl.BlockSpec((blk, d), lambda i: (i, 0)),
        ],
        out_shape=[
            jax.ShapeDtypeStruct((n, d), jnp.float32),
            jax.ShapeDtypeStruct((n, d), jnp.float32),
        ],
    )(nf, w1a, w1b, c1)


def _edge_body(pre_ref, ef_ref, w1c_ref, w2_ref, c2_ref, out_ref):
    ef = ef_ref[...]
    h = pre_ref[...] + jnp.dot(ef, w1c_ref[...], preferred_element_type=jnp.float32)
    h = _silu(h)
    u = jnp.dot(h, w2_ref[...], preferred_element_type=jnp.float32) + c2_ref[...]
    out_ref[...] = _silu(u + ef)


def _edge_mlp(pre, ef, w1c, w2, c2, blk):
    e, d = ef.shape
    grid = e // blk
    return pl.pallas_call(
        _edge_body,
        grid=(grid,),
        in_specs=[
            pl.BlockSpec((blk, d), lambda i: (i, 0)),
            pl.BlockSpec((blk, d), lambda i: (i, 0)),
            pl.BlockSpec((d, d), lambda i: (0, 0)),
            pl.BlockSpec((d, d), lambda i: (0, 0)),
            pl.BlockSpec((1, d), lambda i: (0, 0)),
        ],
        out_specs=pl.BlockSpec((blk, d), lambda i: (i, 0)),
        out_shape=jax.ShapeDtypeStruct((e, d), jnp.float32),
    )(pre, ef, w1c, w2, c2)


def _node_body(nf_ref, agg_ref, cnt_ref, v1a_ref, v1b_ref, c1_ref, v2_ref,
               c2_ref, out_ref):
    nf = nf_ref[...]
    agg = agg_ref[...]
    cnt = jnp.maximum(cnt_ref[...], 1.0)
    h_agg = agg / cnt
    h = (jnp.dot(nf, v1a_ref[...], preferred_element_type=jnp.float32)
         + jnp.dot(h_agg, v1b_ref[...], preferred_element_type=jnp.float32)
         + c1_ref[...])
    h = _silu(h)
    u = jnp.dot(h, v2_ref[...], preferred_element_type=jnp.float32) + c2_ref[...]
    out_ref[...] = _silu(u + nf)


def _node_mlp(nf, agg, cnt, v1a, v1b, c1, v2, c2, blk):
    n, d = nf.shape
    grid = n // blk
    return pl.pallas_call(
        _node_body,
        grid=(grid,),
        in_specs=[
            pl.BlockSpec((blk, d), lambda i: (i, 0)),
            pl.BlockSpec((blk, d), lambda i: (i, 0)),
            pl.BlockSpec((blk, 1), lambda i: (i, 0)),
            pl.BlockSpec((d, d), lambda i: (0, 0)),
            pl.BlockSpec((d, d), lambda i: (0, 0)),
            pl.BlockSpec((1, d), lambda i: (0, 0)),
            pl.BlockSpec((d, d), lambda i: (0, 0)),
            pl.BlockSpec((1, d), lambda i: (0, 0)),
        ],
        out_specs=pl.BlockSpec((blk, d), lambda i: (i, 0)),
        out_shape=jax.ShapeDtypeStruct((n, d), jnp.float32),
    )(nf, agg, cnt, v1a, v1b, c1, v2, c2)


def kernel(node_feats, edge_feats, params, edge_index):
    n, d = node_feats.shape
    e = edge_feats.shape[0]
    src = edge_index[0]
    dst = edge_index[1]

    ew1, ec1, ew2, ec2 = _fold(params['edge'])
    nw1, nc1, nw2, nc2 = _fold(params['node'])
    w1a, w1b, w1c = ew1[:d], ew1[d:2 * d], ew1[2 * d:]
    v1a, v1b = nw1[:d], nw1[d:2 * d]

    psrc, pdst = _node_tables(node_feats, w1a, w1b, ec1.reshape(1, d), blk=2000)

    pre = jnp.take(psrc, src, axis=0) + jnp.take(pdst, dst, axis=0)

    updated_edges = _edge_mlp(pre, edge_feats, w1c, ew2, ec2.reshape(1, d), blk=4000)

    agg = jax.ops.segment_sum(updated_edges, dst, num_segments=n)
    cnt = jax.ops.segment_sum(jnp.ones((e, 1), jnp.float32), dst, num_segments=n)

    updated_nodes = _node_mlp(node_feats, agg, cnt, v1a, v1b,
                              nc1.reshape(1, d), nw2, nc2.reshape(1, d), blk=2000)
    return (updated_nodes, updated_edges)


# speedup vs baseline: 2.0459x; 1.8274x over previous
import functools

import jax
import jax.numpy as jnp
from jax import lax
from jax.experimental import pallas as pl
from jax.experimental.pallas import tpu as pltpu
from jax.experimental.pallas import tpu_sc as plsc

EPS = 1e-5

_SC_CORES = 2
_SC_SUBCORES = 16
_NW = _SC_CORES * _SC_SUBCORES


def _gather_pre(psrc, pdst, src, dst, chunk=80):
    n, d = psrc.shape
    e = src.shape[0]
    per_w = e // _NW
    steps = per_w // chunk
    assert per_w * _NW == e and steps * chunk == per_w and chunk % 8 == 0
    mesh = plsc.VectorSubcoreMesh(core_axis_name="c", subcore_axis_name="s")

    @functools.partial(
        pl.kernel, mesh=mesh,
        out_type=jax.ShapeDtypeStruct((e, d), jnp.float32),
        scratch_types=[
            pltpu.VMEM((chunk,), jnp.int32),
            pltpu.VMEM((chunk,), jnp.int32),
            pltpu.VMEM((chunk, d), jnp.float32),
            pltpu.SemaphoreType.DMA,
        ],
    )
    def k(psrc_hbm, pdst_hbm, src_hbm, dst_hbm, out_hbm, idx_s, idx_d, buf, sem):
        wid = lax.axis_index("s") * _SC_CORES + lax.axis_index("c")
        base = wid * per_w

        @pl.loop(0, steps)
        def _(i):
            off = base + i * chunk
            pltpu.sync_copy(src_hbm.at[pl.ds(off, chunk)], idx_s)
            pltpu.sync_copy(dst_hbm.at[pl.ds(off, chunk)], idx_d)
            pltpu.async_copy(psrc_hbm.at[idx_s], buf, sem).wait()
            pltpu.async_copy(pdst_hbm.at[idx_d], buf, sem, add=True).wait()
            pltpu.sync_copy(buf, out_hbm.at[pl.ds(off, chunk)])

    return k(psrc, pdst, src, dst)


def _fold(p):
    s1 = p['g1'] / jnp.sqrt(p['v1'] + EPS)
    t1 = p['be1'] - p['m1'] * s1
    W1 = p['W1'] * s1[None, :]
    c1 = p['b1'] * s1 + t1
    s2 = p['g2'] / jnp.sqrt(p['v2'] + EPS)
    t2 = p['be2'] - p['m2'] * s2
    W2 = p['W2'] * s2[None, :]
    c2 = p['b2'] * s2 + t2
    return W1, c1, W2, c2


def _silu(x):
    return x * jax.nn.sigmoid(x)


def _tables_body(nf_ref, w1a_ref, w1b_ref, c1_ref, psrc_ref, pdst_ref):
    nf = nf_ref[...]
    psrc_ref[...] = jnp.dot(nf, w1a_ref[...], preferred_element_type=jnp.float32)
    pdst_ref[...] = jnp.dot(nf, w1b_ref[...], preferred_element_type=jnp.float32) + c1_ref[...]


def _node_tables(nf, w1a, w1b, c1, blk):
    n, d = nf.shape
    grid = n // blk
    return pl.pallas_call(
        _tables_body,
        grid=(grid,),
        in_specs=[
            pl.BlockSpec((blk, d), lambda i: (i, 0)),
            pl.BlockSpec((d, d), lambda i: (0, 0)),
            pl.BlockSpec((d, d), lambda i: (0, 0)),
            pl.BlockSpec((1, d), lambda i: (0, 0)),
        ],
        out_specs=[
            pl.BlockSpec((blk, d), lambda i: (i, 0)),
            pl.BlockSpec((blk, d), lambda i: (i, 0)),
        ],
        out_shape=[
            jax.ShapeDtypeStruct((n, d), jnp.float32),
            jax.ShapeDtypeStruct((n, d), jnp.float32),
        ],
    )(nf, w1a, w1b, c1)


def _edge_body(pre_ref, ef_ref, w1c_ref, w2_ref, c2_ref, out_ref):
    ef = ef_ref[...]
    h = pre_ref[...] + jnp.dot(ef, w1c_ref[...], preferred_element_type=jnp.float32)
    h = _silu(h)
    u = jnp.dot(h, w2_ref[...], preferred_element_type=jnp.float32) + c2_ref[...]
    out_ref[...] = _silu(u + ef)


def _edge_mlp(pre, ef, w1c, w2, c2, blk):
    e, d = ef.shape
    grid = e // blk
    return pl.pallas_call(
        _edge_body,
        grid=(grid,),
        in_specs=[
            pl.BlockSpec((blk, d), lambda i: (i, 0)),
            pl.BlockSpec((blk, d), lambda i: (i, 0)),
            pl.BlockSpec((d, d), lambda i: (0, 0)),
            pl.BlockSpec((d, d), lambda i: (0, 0)),
            pl.BlockSpec((1, d), lambda i: (0, 0)),
        ],
        out_specs=pl.BlockSpec((blk, d), lambda i: (i, 0)),
        out_shape=jax.ShapeDtypeStruct((e, d), jnp.float32),
    )(pre, ef, w1c, w2, c2)


def _node_body(nf_ref, agg_ref, cnt_ref, v1a_ref, v1b_ref, c1_ref, v2_ref,
               c2_ref, out_ref):
    nf = nf_ref[...]
    agg = agg_ref[...]
    cnt = jnp.maximum(cnt_ref[...], 1.0)
    h_agg = agg / cnt
    h = (jnp.dot(nf, v1a_ref[...], preferred_element_type=jnp.float32)
         + jnp.dot(h_agg, v1b_ref[...], preferred_element_type=jnp.float32)
         + c1_ref[...])
    h = _silu(h)
    u = jnp.dot(h, v2_ref[...], preferred_element_type=jnp.float32) + c2_ref[...]
    out_ref[...] = _silu(u + nf)


def _node_mlp(nf, agg, cnt, v1a, v1b, c1, v2, c2, blk):
    n, d = nf.shape
    grid = n // blk
    return pl.pallas_call(
        _node_body,
        grid=(grid,),
        in_specs=[
            pl.BlockSpec((blk, d), lambda i: (i, 0)),
            pl.BlockSpec((blk, d), lambda i: (i, 0)),
            pl.BlockSpec((blk, 1), lambda i: (i, 0)),
            pl.BlockSpec((d, d), lambda i: (0, 0)),
            pl.BlockSpec((d, d), lambda i: (0, 0)),
            pl.BlockSpec((1, d), lambda i: (0, 0)),
            pl.BlockSpec((d, d), lambda i: (0, 0)),
            pl.BlockSpec((1, d), lambda i: (0, 0)),
        ],
        out_specs=pl.BlockSpec((blk, d), lambda i: (i, 0)),
        out_shape=jax.ShapeDtypeStruct((n, d), jnp.float32),
    )(nf, agg, cnt, v1a, v1b, c1, v2, c2)


def kernel(node_feats, edge_feats, params, edge_index):
    n, d = node_feats.shape
    e = edge_feats.shape[0]
    src = edge_index[0]
    dst = edge_index[1]

    ew1, ec1, ew2, ec2 = _fold(params['edge'])
    nw1, nc1, nw2, nc2 = _fold(params['node'])
    w1a, w1b, w1c = ew1[:d], ew1[d:2 * d], ew1[2 * d:]
    v1a, v1b = nw1[:d], nw1[d:2 * d]

    psrc, pdst = _node_tables(node_feats, w1a, w1b, ec1.reshape(1, d), blk=2000)

    pre = _gather_pre(psrc, pdst, src, dst)

    updated_edges = _edge_mlp(pre, edge_feats, w1c, ew2, ec2.reshape(1, d), blk=4000)

    agg = jax.ops.segment_sum(updated_edges, dst, num_segments=n)
    cnt = jax.ops.segment_sum(jnp.ones((e, 1), jnp.float32), dst, num_segments=n)

    updated_nodes = _node_mlp(node_feats, agg, cnt, v1a, v1b,
                              nc1.reshape(1, d), nw2, nc2.reshape(1, d), blk=2000)
    return (updated_nodes, updated_edges)


# speedup vs baseline: 4.0607x; 1.9848x over previous
import dataclasses
import functools

import jax
import jax.numpy as jnp
from jax import lax
from jax.experimental import pallas as pl
from jax.experimental.pallas import tpu as pltpu
from jax.experimental.pallas import tpu_sc as plsc

EPS = 1e-5

_SC_CORES = 2
_SC_SUBCORES = 16
_NW = _SC_CORES * _SC_SUBCORES


def _gather_pre(psrc, pdst, src, dst, chunk=80):
    n, d = psrc.shape
    e = src.shape[0]
    per_w = e // _NW
    steps = per_w // chunk
    assert per_w * _NW == e and steps * chunk == per_w and chunk % 8 == 0
    mesh = plsc.VectorSubcoreMesh(core_axis_name="c", subcore_axis_name="s")

    @functools.partial(
        pl.kernel, mesh=mesh,
        out_type=jax.ShapeDtypeStruct((e, d), jnp.float32),
        scratch_types=[
            pltpu.VMEM((chunk,), jnp.int32),
            pltpu.VMEM((chunk,), jnp.int32),
            pltpu.VMEM((chunk, d), jnp.float32),
            pltpu.SemaphoreType.DMA,
        ],
    )
    def k(psrc_hbm, pdst_hbm, src_hbm, dst_hbm, out_hbm, idx_s, idx_d, buf, sem):
        wid = lax.axis_index("s") * _SC_CORES + lax.axis_index("c")
        base = wid * per_w

        @pl.loop(0, steps)
        def _(i):
            off = base + i * chunk
            pltpu.sync_copy(src_hbm.at[pl.ds(off, chunk)], idx_s)
            pltpu.sync_copy(dst_hbm.at[pl.ds(off, chunk)], idx_d)
            pltpu.async_copy(psrc_hbm.at[idx_s], buf, sem).wait()
            pltpu.async_copy(pdst_hbm.at[idx_d], buf, sem, add=True).wait()
            pltpu.sync_copy(buf, out_hbm.at[pl.ds(off, chunk)])

    return k(psrc, pdst, src, dst)


def _fold(p):
    s1 = p['g1'] / jnp.sqrt(p['v1'] + EPS)
    t1 = p['be1'] - p['m1'] * s1
    W1 = p['W1'] * s1[None, :]
    c1 = p['b1'] * s1 + t1
    s2 = p['g2'] / jnp.sqrt(p['v2'] + EPS)
    t2 = p['be2'] - p['m2'] * s2
    W2 = p['W2'] * s2[None, :]
    c2 = p['b2'] * s2 + t2
    return W1, c1, W2, c2


def _silu(x):
    return x * jax.nn.sigmoid(x)


def _tables_body(nf_ref, w1a_ref, w1b_ref, c1_ref, psrc_ref, pdst_ref):
    nf = nf_ref[...]
    psrc_ref[...] = jnp.dot(nf, w1a_ref[...], preferred_element_type=jnp.float32)
    pdst_ref[...] = jnp.dot(nf, w1b_ref[...], preferred_element_type=jnp.float32) + c1_ref[...]


def _node_tables(nf, w1a, w1b, c1, blk):
    n, d = nf.shape
    grid = n // blk
    return pl.pallas_call(
        _tables_body,
        grid=(grid,),
        in_specs=[
            pl.BlockSpec((blk, d), lambda i: (i, 0)),
            pl.BlockSpec((d, d), lambda i: (0, 0)),
            pl.BlockSpec((d, d), lambda i: (0, 0)),
            pl.BlockSpec((1, d), lambda i: (0, 0)),
        ],
        out_specs=[
            pl.BlockSpec((blk, d), lambda i: (i, 0)),
            pl.BlockSpec((blk, d), lambda i: (i, 0)),
        ],
        out_shape=[
            jax.ShapeDtypeStruct((n, d), jnp.float32),
            jax.ShapeDtypeStruct((n, d), jnp.float32),
        ],
    )(nf, w1a, w1b, c1)


def _seg_pad(n, chunk):
    rows_per_sub = -(-n // (chunk * _SC_SUBCORES)) * chunk
    return rows_per_sub, rows_per_sub * _SC_SUBCORES


def _scatter_agg(upd, dst, n, chunk=128):
    e, d = upd.shape
    assert e % chunk == 0 and chunk % 8 == 0
    stage = 64
    rows_per_sub, npad = _seg_pad(n, chunk)
    sub_steps = rows_per_sub // stage
    assert sub_steps * stage == rows_per_sub
    mesh = plsc.VectorSubcoreMesh(core_axis_name="c", subcore_axis_name="s")
    zero_rows = jnp.zeros((stage, d), jnp.float32)
    row_iota = jnp.arange(npad, dtype=jnp.int32)

    @functools.partial(
        pl.kernel, mesh=mesh,
        out_type=jax.ShapeDtypeStruct((_SC_CORES * npad, d), jnp.float32),
        scratch_types=[
            pltpu.VMEM((stage,), jnp.int32),
            pltpu.VMEM((stage, d), jnp.float32),
            pltpu.VMEM_SHARED((npad, d), jnp.float32),
            pltpu.SemaphoreType.DMA,
        ],
    )
    def k(upd_hbm, dst_hbm, zrow_hbm, iota_hbm, agg_hbm,
          idx_r, buf, spmem_agg, sem):
        cid = lax.axis_index("c")
        sid = lax.axis_index("s")
        rbase = sid * rows_per_sub

        pltpu.sync_copy(zrow_hbm, buf)

        @pl.loop(0, sub_steps)
        def _(j):
            r = rbase + j * stage
            pltpu.sync_copy(iota_hbm.at[pl.ds(r, stage)], idx_r)
            pltpu.sync_copy(buf, spmem_agg.at[idx_r])

        plsc.subcore_barrier()

        def acc_body(upd_vmem, i_vmem):
            pltpu.sync_copy(upd_vmem, spmem_agg.at[i_vmem.at[0]], add=True)

        pltpu.emit_pipeline(
            acc_body,
            grid=(e // chunk,),
            in_specs=[
                pl.BlockSpec((chunk, d), lambda i: (i, 0)),
                pl.BlockSpec((1, chunk), lambda i: (0, i)),
            ],
            out_specs=[],
            core_axis_name=("c", "s"),
            dimension_semantics=(pltpu.PARALLEL,),
        )(upd_hbm, dst_hbm)

        plsc.subcore_barrier()

        obase = cid * npad + rbase

        @pl.loop(0, sub_steps)
        def _(j):
            r = rbase + j * stage
            o = obase + j * stage
            pltpu.sync_copy(iota_hbm.at[pl.ds(r, stage)], idx_r)
            pltpu.async_copy(spmem_agg.at[idx_r], buf, sem).wait()
            pltpu.sync_copy(buf, agg_hbm.at[pl.ds(o, stage)])

    agg2 = k(upd, dst.reshape(1, e), zero_rows, row_iota)
    return agg2.reshape(_SC_CORES, npad, d)


def _scatter_cnt(dst, n, chunk=80):
    e = dst.shape[0]
    per_w = e // _NW
    steps = per_w // chunk
    assert per_w * _NW == e and steps * chunk == per_w
    assert chunk % 16 == 0 and chunk % 8 == 0
    _, npad = _seg_pad(n, 128)
    mesh = plsc.VectorSubcoreMesh(core_axis_name="c", subcore_axis_name="s")
    zeros_n = jnp.zeros((npad,), jnp.float32)

    cp = pltpu.CompilerParams()
    if "needs_layout_passes" in pltpu.CompilerParams.__dataclass_fields__:
        cp = dataclasses.replace(cp, needs_layout_passes=False)

    @functools.partial(
        pl.kernel, mesh=mesh,
        out_type=jax.ShapeDtypeStruct((_NW * npad,), jnp.float32),
        scratch_types=[
            pltpu.VMEM((chunk,), jnp.int32),
            pltpu.VMEM((npad,), jnp.float32),
        ],
        compiler_params=cp,
    )
    def k(dst_hbm, zeros_hbm, cnt_hbm, idx_v, cnt_local):
        cid = lax.axis_index("c")
        sid = lax.axis_index("s")
        wid = sid * _SC_CORES + cid
        base = wid * per_w
        ones16 = jnp.ones((16,), jnp.float32)

        pltpu.sync_copy(zeros_hbm, cnt_local)

        @pl.loop(0, steps)
        def _(i):
            pltpu.sync_copy(dst_hbm.at[pl.ds(base + i * chunk, chunk)], idx_v)

            @pl.loop(0, chunk // 16)
            def _(j):
                iv = idx_v[pl.ds(j * 16, 16)]
                plsc.addupdate_scatter(cnt_local, [iv], ones16)

        pltpu.sync_copy(cnt_local, cnt_hbm.at[pl.ds(wid * npad, npad)])

    cnt2 = k(dst, zeros_n)
    return cnt2.reshape(_NW, npad).T


def _edge_body(pre_ref, ef_ref, w1c_ref, w2_ref, c2_ref, out_ref):
    ef = ef_ref[...]
    h = pre_ref[...] + jnp.dot(ef, w1c_ref[...], preferred_element_type=jnp.float32)
    h = _silu(h)
    u = jnp.dot(h, w2_ref[...], preferred_element_type=jnp.float32) + c2_ref[...]
    out_ref[...] = _silu(u + ef)


def _edge_mlp(pre, ef, w1c, w2, c2, blk):
    e, d = ef.shape
    grid = e // blk
    return pl.pallas_call(
        _edge_body,
        grid=(grid,),
        in_specs=[
            pl.BlockSpec((blk, d), lambda i: (i, 0)),
            pl.BlockSpec((blk, d), lambda i: (i, 0)),
            pl.BlockSpec((d, d), lambda i: (0, 0)),
            pl.BlockSpec((d, d), lambda i: (0, 0)),
            pl.BlockSpec((1, d), lambda i: (0, 0)),
        ],
        out_specs=pl.BlockSpec((blk, d), lambda i: (i, 0)),
        out_shape=jax.ShapeDtypeStruct((e, d), jnp.float32),
    )(pre, ef, w1c, w2, c2)


def _node_body(nf_ref, agg_ref, cnt_ref, v1a_ref, v1b_ref, c1_ref, v2_ref,
               c2_ref, out_ref):
    nf = nf_ref[...]
    agg = agg_ref[0] + agg_ref[1]
    cnt = jnp.maximum(jnp.sum(cnt_ref[...], axis=1), 1.0)
    h_agg = agg / cnt[:, None]
    h = (jnp.dot(nf, v1a_ref[...], preferred_element_type=jnp.float32)
         + jnp.dot(h_agg, v1b_ref[...], preferred_element_type=jnp.float32)
         + c1_ref[...])
    h = _silu(h)
    u = jnp.dot(h, v2_ref[...], preferred_element_type=jnp.float32) + c2_ref[...]
    out_ref[...] = _silu(u + nf)


def _node_mlp(nf, agg, cnt, v1a, v1b, c1, v2, c2, blk):
    n, d = nf.shape
    grid = n // blk
    return pl.pallas_call(
        _node_body,
        grid=(grid,),
        in_specs=[
            pl.BlockSpec((blk, d), lambda i: (i, 0)),
            pl.BlockSpec((2, blk, d), lambda i: (0, i, 0)),
            pl.BlockSpec((blk, _NW), lambda i: (i, 0)),
            pl.BlockSpec((d, d), lambda i: (0, 0)),
            pl.BlockSpec((d, d), lambda i: (0, 0)),
            pl.BlockSpec((1, d), lambda i: (0, 0)),
            pl.BlockSpec((d, d), lambda i: (0, 0)),
            pl.BlockSpec((1, d), lambda i: (0, 0)),
        ],
        out_specs=pl.BlockSpec((blk, d), lambda i: (i, 0)),
        out_shape=jax.ShapeDtypeStruct((n, d), jnp.float32),
    )(nf, agg, cnt, v1a, v1b, c1, v2, c2)


def kernel(node_feats, edge_feats, params, edge_index):
    n, d = node_feats.shape
    e = edge_feats.shape[0]
    src = edge_index[0]
    dst = edge_index[1]

    ew1, ec1, ew2, ec2 = _fold(params['edge'])
    nw1, nc1, nw2, nc2 = _fold(params['node'])
    w1a, w1b, w1c = ew1[:d], ew1[d:2 * d], ew1[2 * d:]
    v1a, v1b = nw1[:d], nw1[d:2 * d]

    psrc, pdst = _node_tables(node_feats, w1a, w1b, ec1.reshape(1, d), blk=2000)

    pre = _gather_pre(psrc, pdst, src, dst)

    updated_edges = _edge_mlp(pre, edge_feats, w1c, ew2, ec2.reshape(1, d), blk=4000)

    cnt_part = _scatter_cnt(dst, n)
    agg_part = _scatter_agg(updated_edges, dst, n)

    updated_nodes = _node_mlp(node_feats, agg_part, cnt_part, v1a, v1b,
                              nc1.reshape(1, d), nw2, nc2.reshape(1, d), blk=2000)
    return (updated_nodes, updated_edges)
